# baseline (speedup 1.0000x reference)
import functools
import math

import jax
import jax.numpy as jnp
from jax import lax
from jax.experimental import pallas as pl
from jax.experimental.pallas import tpu as pltpu
from jax.experimental.pallas import tpu_sc as plsc

D_MODEL = 64
SCALE = math.sqrt(float(D_MODEL))

_NC = 2
_NS = 16
_NW = _NC * _NS

_LANES = 16
_IDX_W = 128
_CHUNK = 256
_GPC = _CHUNK // _IDX_W
_ROW_W = 128


def _body(x_hbm, tbl_hbm, out_hbm, idx_v, row_v, half_v, rows_v0, rows_v1,
          tile_v, sem_i, sem_g, sem_s):
    rows_bufs = (rows_v0, rows_v1)
    wid = lax.axis_index("s") * _NC + lax.axis_index("c")
    n_s = x_hbm.shape[0]
    ncb = x_hbm.shape[1] // _CHUNK
    upw = n_s * ncb // _NW
    u0 = wid * upw

    def idx_src(u):
        s = u // ncb
        cb = u % ncb
        return x_hbm.at[s, pl.ds(cb * _CHUNK, _CHUNK)]

    def out_dst(u):
        s = u // ncb
        cb = u % ncb
        return out_hbm.at[s, :, pl.ds(cb * _CHUNK, _CHUNK)]

    def compute_meta(b):
        for i in range(_CHUNK // _LANES):
            sl = pl.ds(i * _LANES, _LANES)
            v = idx_v[b, sl]
            row_v[b, sl] = v >> 1
            half_v[b, sl] = (v & 1) * D_MODEL

    def fire_gathers(b):
        for j in range(_GPC):
            sl = pl.ds(j * _IDX_W, _IDX_W)
            pltpu.async_copy(tbl_hbm.at[row_v.at[b, sl]],
                             rows_bufs[b].at[sl], sem_g)

    def wait_gathers(b):
        pltpu.make_async_copy(tbl_hbm.at[pl.ds(0, _CHUNK)],
                              rows_bufs[b], sem_g).wait()

    def transpose(p):
        rows_p = rows_bufs[p]
        base = lax.iota(jnp.int32, _LANES)
        for v in range(_CHUNK // _LANES):
            sl = pl.ds(v * _LANES, _LANES)
            rvec = base + (v * _LANES)
            hvec = half_v[p, sl]

            def col_body(c, carry, rvec=rvec, hvec=hvec, sl=sl):
                col = hvec + c
                vals = plsc.load_gather(rows_p, [rvec, col])
                tile_v[p, c, sl] = vals
                return carry

            lax.fori_loop(0, D_MODEL, col_body, 0)

    def fire_out(u, p):
        pltpu.async_copy(tile_v.at[p], out_dst(u), sem_s)

    def drain_out(u, p):
        pltpu.make_async_copy(tile_v.at[p], out_dst(u), sem_s).wait()

    def do_unit(k, p, no_drain, has_next, has_idx2):
        q = 1 - p
        u = u0 + k
        if has_next:
            pltpu.make_async_copy(idx_src(u + 1), idx_v.at[q], sem_i).wait()
            compute_meta(q)
            fire_gathers(q)
        if has_idx2:
            pltpu.async_copy(idx_src(u + 2), idx_v.at[p], sem_i)
        wait_gathers(p)
        if not no_drain:
            drain_out(u - 2, p)
        transpose(p)
        fire_out(u, p)

    pltpu.sync_copy(idx_src(u0), idx_v.at[0])
    compute_meta(0)
    fire_gathers(0)
    pltpu.async_copy(idx_src(u0 + 1), idx_v.at[1], sem_i)

    do_unit(0, 0, True, True, True)
    do_unit(1, 1, True, True, True)

    def pair(kk, carry):
        k = 2 * kk
        do_unit(k, 0, False, True, True)
        do_unit(k + 1, 1, False, True, True)
        return carry
    lax.fori_loop(1, (upw - 4) // 2 + 1, pair, 0)

    do_unit(upw - 2, 0, False, True, False)
    do_unit(upw - 1, 1, False, False, False)
    drain_out(u0 + upw - 2, 0)
    drain_out(u0 + upw - 1, 1)


def kernel(x, table):
    S0, S1 = x.shape
    V, D = table.shape
    xT = x.T.astype(jnp.int32)
    tbl = (table * jnp.float32(SCALE)).reshape(V // 2, 2 * D)

    fn = functools.partial(
        pl.kernel,
        out_type=jax.ShapeDtypeStruct((S1, D_MODEL, S0), jnp.float32),
        mesh=plsc.VectorSubcoreMesh(core_axis_name="c", subcore_axis_name="s"),
        scratch_types=[
            pltpu.VMEM((2, _CHUNK), jnp.int32),
            pltpu.VMEM((2, _CHUNK), jnp.int32),
            pltpu.VMEM((2, _CHUNK), jnp.int32),
            pltpu.VMEM((_CHUNK, _ROW_W), jnp.float32),
            pltpu.VMEM((_CHUNK, _ROW_W), jnp.float32),
            pltpu.VMEM((2, D_MODEL, _CHUNK), jnp.float32),
            pltpu.SemaphoreType.DMA,
            pltpu.SemaphoreType.DMA,
            pltpu.SemaphoreType.DMA,
        ],
        compiler_params=pltpu.CompilerParams(use_tc_tiling_on_sc=False,
                                             needs_layout_passes=False),
    )(_body)
    outT = fn(xT, tbl)
    return outT.transpose(2, 0, 1)


# speedup vs baseline: 1.0006x; 1.0006x over previous
import functools
import math

import jax
import jax.numpy as jnp
from jax import lax
from jax.experimental import pallas as pl
from jax.experimental.pallas import tpu as pltpu
from jax.experimental.pallas import tpu_sc as plsc

D_MODEL = 64
SCALE = math.sqrt(float(D_MODEL))

_NC = 2
_NS = 16
_NW = _NC * _NS

_LANES = 16
_IDX_W = 128
_CHUNK = 256
_GPC = _CHUNK // _IDX_W
_ROW_W = 128


def _body(x_hbm, tbl_hbm, out_hbm, idx_v, row_v, half_v, rows_v0, rows_v1,
          tile_v, sem_i, sem_g, sem_s):
    rows_bufs = (rows_v0, rows_v1)
    wid = lax.axis_index("s") * _NC + lax.axis_index("c")
    n_s = x_hbm.shape[0]
    ncb = x_hbm.shape[1] // _CHUNK
    upw = n_s * ncb // _NW
    u0 = wid * upw

    def idx_src(u):
        s = u // ncb
        cb = u % ncb
        return x_hbm.at[s, pl.ds(cb * _CHUNK, _CHUNK)]

    def out_dst(u):
        s = u // ncb
        cb = u % ncb
        return out_hbm.at[s, :, pl.ds(cb * _CHUNK, _CHUNK)]

    def compute_meta(b):
        for i in range(_CHUNK // _LANES):
            sl = pl.ds(i * _LANES, _LANES)
            v = idx_v[b, sl]
            row_v[b, sl] = v >> 1
            half_v[b, sl] = (v & 1) * D_MODEL

    def fire_gathers(b):
        for j in range(_GPC):
            sl = pl.ds(j * _IDX_W, _IDX_W)
            pltpu.async_copy(tbl_hbm.at[row_v.at[b, sl]],
                             rows_bufs[b].at[sl], sem_g)

    def wait_gathers(b):
        pltpu.make_async_copy(tbl_hbm.at[pl.ds(0, _CHUNK)],
                              rows_bufs[b], sem_g).wait()

    def transpose(p):
        rows_p = rows_bufs[p]
        base = lax.iota(jnp.int32, _LANES)

        def v_body(v, carry):
            sl = pl.ds(v * _LANES, _LANES)
            rvec = base + v * _LANES
            hvec = half_v[p, sl]
            for c in range(D_MODEL):
                vals = plsc.load_gather(rows_p, [rvec, hvec + c])
                tile_v[p, c, sl] = vals
            return carry

        lax.fori_loop(0, _CHUNK // _LANES, v_body, 0)

    def fire_out(u, p):
        pltpu.async_copy(tile_v.at[p], out_dst(u), sem_s)

    def drain_out(u, p):
        pltpu.make_async_copy(tile_v.at[p], out_dst(u), sem_s).wait()

    def do_unit(k, p, no_drain, has_next, has_idx2):
        q = 1 - p
        u = u0 + k
        if has_next:
            pltpu.make_async_copy(idx_src(u + 1), idx_v.at[q], sem_i).wait()
            compute_meta(q)
            fire_gathers(q)
        if has_idx2:
            pltpu.async_copy(idx_src(u + 2), idx_v.at[p], sem_i)
        wait_gathers(p)
        if not no_drain:
            drain_out(u - 2, p)
        transpose(p)
        fire_out(u, p)

    pltpu.sync_copy(idx_src(u0), idx_v.at[0])
    compute_meta(0)
    fire_gathers(0)
    pltpu.async_copy(idx_src(u0 + 1), idx_v.at[1], sem_i)

    do_unit(0, 0, True, True, True)
    do_unit(1, 1, True, True, True)

    def pair(kk, carry):
        k = 2 * kk
        do_unit(k, 0, False, True, True)
        do_unit(k + 1, 1, False, True, True)
        return carry
    lax.fori_loop(1, (upw - 4) // 2 + 1, pair, 0)

    do_unit(upw - 2, 0, False, True, False)
    do_unit(upw - 1, 1, False, False, False)
    drain_out(u0 + upw - 2, 0)
    drain_out(u0 + upw - 1, 1)


def kernel(x, table):
    S0, S1 = x.shape
    V, D = table.shape
    xT = x.T.astype(jnp.int32)
    tbl = (table * jnp.float32(SCALE)).reshape(V // 2, 2 * D)

    fn = functools.partial(
        pl.kernel,
        out_type=jax.ShapeDtypeStruct((S1, D_MODEL, S0), jnp.float32),
        mesh=plsc.VectorSubcoreMesh(core_axis_name="c", subcore_axis_name="s"),
        scratch_types=[
            pltpu.VMEM((2, _CHUNK), jnp.int32),
            pltpu.VMEM((2, _CHUNK), jnp.int32),
            pltpu.VMEM((2, _CHUNK), jnp.int32),
            pltpu.VMEM((_CHUNK, _ROW_W), jnp.float32),
            pltpu.VMEM((_CHUNK, _ROW_W), jnp.float32),
            pltpu.VMEM((2, D_MODEL, _CHUNK), jnp.float32),
            pltpu.SemaphoreType.DMA,
            pltpu.SemaphoreType.DMA,
            pltpu.SemaphoreType.DMA,
        ],
        compiler_params=pltpu.CompilerParams(use_tc_tiling_on_sc=False,
                                             needs_layout_passes=False),
    )(_body)
    outT = fn(xT, tbl)
    return outT.transpose(2, 0, 1)
